# Initial kernel scaffold; baseline (speedup 1.0000x reference)
import jax
import jax.numpy as jnp
from jax.experimental import pallas as pl


def kernel(score, rep_srcs, rep_idx, score_idx):
    raise NotImplementedError("write your pallas kernel here")


# speedup vs baseline: 6.5882x; 6.5882x over previous
import functools

import jax
import jax.numpy as jnp
from jax import lax
from jax.experimental import pallas as pl
from jax.experimental.pallas import tpu as pltpu
from jax.experimental.pallas import tpu_sc as plsc

B = 8
L = 1024
M = L - 2
ROWS = M - 1
NG = 32


def _fill_pad(pad_v, row_v, rev):
    zz = jnp.zeros((16,), jnp.float32)
    for k in range(64):
        pad_v[pl.ds(k * 16, 16)] = zz
    for k in range(64):
        if not rev:
            v = row_v[pl.ds(9 + k * 16, 16)]
        else:
            v = lax.rev(row_v[pl.ds(1015 - k * 16, 16)], (0,))
        pad_v[pl.ds(M + k * 16, 16)] = v
    pad_v[pl.ds(M + 1022, 16)] = zz


def _run_groups(out, b, h, pad_v, obufs, sems):
    handles = [None, None]
    for g in range(NG):
        db = g & 1
        if handles[db] is not None:
            handles[db].wait()
        c0 = h * 512 + g * 16
        ob = obufs[db]
        sj = pad_v[pl.ds(M + c0, 16)]
        aco = 5.0 - 20.0 * sj

        def rowfn(i, run, ob=ob, c0=c0, aco=aco):
            sh = pad_v[pl.ds(ROWS - i + c0, 16)]
            t = sh * 20.0 + aco
            e = jnp.exp(jnp.minimum(-t, 80.0))
            run = run * (1.0 / (1.0 + e))
            ob[i] = run
            return run

        lax.fori_loop(0, ROWS, rowfn, jnp.ones((16,), jnp.float32))
        handles[db] = pltpu.async_copy(ob, out.at[b, :, pl.ds(c0, 16)], sems[db])
    for hd in handles:
        hd.wait()


def _body(score_hbm, sidx_hbm, fwd_hbm, bwd_hbm,
          score_v, idx_v, row_v, pad_v, ob0, ob1, sem0, sem1):
    c = lax.axis_index("c")
    s = lax.axis_index("s")
    b = jnp.where(s < 8, s, s - 8)

    pltpu.sync_copy(score_hbm, score_v)
    pltpu.sync_copy(sidx_hbm.at[b], idx_v)
    for k in range(64):
        iv = idx_v[pl.ds(k * 16, 16)]
        row_v[pl.ds(8 + k * 16, 16)] = plsc.load_gather(score_v, [iv])

    @pl.when(s < 8)
    def _():
        _fill_pad(pad_v, row_v, rev=False)
        _run_groups(fwd_hbm, b, c, pad_v, (ob0, ob1), (sem0, sem1))

    @pl.when(s >= 8)
    def _():
        _fill_pad(pad_v, row_v, rev=True)
        _run_groups(bwd_hbm, b, c, pad_v, (ob0, ob1), (sem0, sem1))


@functools.partial(jax.jit, static_argnames=())
def _gate_net(score, score_idx):
    mesh = plsc.VectorSubcoreMesh(core_axis_name="c", subcore_axis_name="s")
    out_ty = (jax.ShapeDtypeStruct((B, ROWS, M), jnp.float32),
              jax.ShapeDtypeStruct((B, ROWS, M), jnp.float32))
    fn = pl.kernel(
        _body,
        mesh=mesh,
        out_type=out_ty,
        scratch_types=[
            pltpu.VMEM((B * L,), jnp.float32),
            pltpu.VMEM((L,), jnp.int32),
            pltpu.VMEM((8 + L + 8,), jnp.float32),
            pltpu.VMEM((2 * L + 16,), jnp.float32),
            pltpu.VMEM((ROWS, 16), jnp.float32),
            pltpu.VMEM((ROWS, 16), jnp.float32),
            pltpu.SemaphoreType.DMA,
            pltpu.SemaphoreType.DMA,
        ],
        compiler_params=pltpu.CompilerParams(use_tc_tiling_on_sc=False,
                                             needs_layout_passes=False),
    )
    return fn(score, score_idx)


def kernel(score, rep_srcs, rep_idx, score_idx):
    del rep_srcs, rep_idx
    return _gate_net(score, score_idx.astype(jnp.int32))


# speedup vs baseline: 17.8341x; 2.7070x over previous
import functools

import jax
import jax.numpy as jnp
from jax import lax
from jax.experimental import pallas as pl
from jax.experimental.pallas import tpu as pltpu
from jax.experimental.pallas import tpu_sc as plsc

B = 8
L = 1024
M = L - 2
ROWS = M - 1


def _fill_pad(pad_v, row_v, rev):
    zz = jnp.zeros((16,), jnp.float32)
    for k in range(64):
        pad_v[pl.ds(k * 16, 16)] = zz
    for k in range(64):
        if not rev:
            v = row_v[pl.ds(9 + k * 16, 16)]
        else:
            v = lax.rev(row_v[pl.ds(1015 - k * 16, 16)], (0,))
        pad_v[pl.ds(M + k * 16, 16)] = v
    pad_v[pl.ds(M + 1022, 16)] = zz


def _body(score_hbm, sidx_hbm, out_hbm,
          score_v, idx_v, row_v, pad_v, ob0, ob1, sem0, sem1):
    c = lax.axis_index("c")
    s = lax.axis_index("s")
    b = jnp.where(s < 8, s, s - 8)
    dd = jnp.where(s < 8, 0, 1)

    pltpu.sync_copy(score_hbm, score_v)
    pltpu.sync_copy(sidx_hbm.at[b], idx_v)
    for k in range(64):
        iv = idx_v[pl.ds(k * 16, 16)]
        row_v[pl.ds(8 + k * 16, 16)] = plsc.load_gather(score_v, [iv])

    @pl.when(s < 8)
    def _():
        _fill_pad(pad_v, row_v, rev=False)

    @pl.when(s >= 8)
    def _():
        _fill_pad(pad_v, row_v, rev=True)

    def compute_group(c0, ob):
        sj = pad_v[pl.ds(M + c0, 16)]
        aco = 5.0 - 20.0 * sj

        def sig(off):
            t = pad_v[pl.ds(off, 16)] * 20.0 + aco
            return 1.0 / (1.0 + jnp.exp(jnp.minimum(-t, 80.0)))

        def blockfn(it, run):
            i0 = it * 8
            offb = ROWS + c0 - i0
            gs = [sig(offb - u) for u in range(8)]
            a1 = gs[0] * gs[1]
            a3 = gs[2] * gs[3]
            a5 = gs[4] * gs[5]
            a7 = gs[6] * gs[7]
            b3 = a1 * a3
            p = [gs[0], a1, a1 * gs[2], b3, b3 * gs[4], b3 * a5,
                 b3 * (a5 * gs[6]), b3 * (a5 * a7)]
            for u in range(8):
                ob[i0 + u] = run * p[u]
            return run * p[7]

        run = lax.fori_loop(0, ROWS // 8, blockfn, jnp.ones((16,), jnp.float32))
        for i in range(8 * (ROWS // 8), ROWS):
            run = run * sig(ROWS + c0 - i)
            ob[i] = run

    def dma_start(ob, c0, sem):
        pltpu.async_copy(ob, out_hbm.at[dd, b, :, pl.ds(c0, 16)], sem)

    def dma_drain(ob, c0, sem):
        pltpu.make_async_copy(
            ob, out_hbm.at[dd, b, :, pl.ds(c0, 16)], sem).wait()

    def tbody(t, carry):
        c0a = (4 * t + c) * 16
        c0b = (4 * t + 2 + c) * 16

        @pl.when(t > 0)
        def _():
            dma_drain(ob0, c0a, sem0)

        compute_group(c0a, ob0)
        dma_start(ob0, c0a, sem0)

        @pl.when(t > 0)
        def _():
            dma_drain(ob1, c0b, sem1)

        compute_group(c0b, ob1)
        dma_start(ob1, c0b, sem1)
        return carry

    lax.fori_loop(0, 16, tbody, jnp.int32(0))
    dma_drain(ob0, (60 + c) * 16, sem0)
    dma_drain(ob1, (62 + c) * 16, sem1)


@functools.partial(jax.jit, static_argnames=())
def _gate_net(score, score_idx):
    mesh = plsc.VectorSubcoreMesh(core_axis_name="c", subcore_axis_name="s")
    out_ty = jax.ShapeDtypeStruct((2, B, ROWS, M), jnp.float32)
    fn = pl.kernel(
        _body,
        mesh=mesh,
        out_type=out_ty,
        scratch_types=[
            pltpu.VMEM((B * L,), jnp.float32),
            pltpu.VMEM((L,), jnp.int32),
            pltpu.VMEM((8 + L + 8,), jnp.float32),
            pltpu.VMEM((2 * L + 16,), jnp.float32),
            pltpu.VMEM((ROWS, 16), jnp.float32),
            pltpu.VMEM((ROWS, 16), jnp.float32),
            pltpu.SemaphoreType.DMA,
            pltpu.SemaphoreType.DMA,
        ],
        compiler_params=pltpu.CompilerParams(use_tc_tiling_on_sc=False,
                                             needs_layout_passes=False),
    )
    out = fn(score, score_idx)
    return out[0], out[1]


def kernel(score, rep_srcs, rep_idx, score_idx):
    del rep_srcs, rep_idx
    return _gate_net(score, score_idx.astype(jnp.int32))


# speedup vs baseline: 22.1891x; 1.2442x over previous
import functools

import jax
import jax.numpy as jnp
from jax import lax
from jax.experimental import pallas as pl
from jax.experimental.pallas import tpu as pltpu
from jax.experimental.pallas import tpu_sc as plsc

B = 8
L = 1024
M = L - 2
ROWS = M - 1


def _fill_pad(pad_v, row_v, rev):
    zz = jnp.zeros((16,), jnp.float32)
    for k in range(64):
        pad_v[pl.ds(k * 16, 16)] = zz
    for k in range(64):
        if not rev:
            v = row_v[pl.ds(9 + k * 16, 16)]
        else:
            v = lax.rev(row_v[pl.ds(1015 - k * 16, 16)], (0,))
        pad_v[pl.ds(M + k * 16, 16)] = v
    pad_v[pl.ds(M + 1022, 16)] = zz


def _body(score_hbm, sidx_hbm, out_hbm,
          score_v, idx_v, row_v, pad_v, ob0, ob1, sem0, sem1):
    c = lax.axis_index("c")
    s = lax.axis_index("s")
    b = jnp.where(s < 8, s, s - 8)
    dd = jnp.where(s < 8, 0, 1)

    pltpu.sync_copy(score_hbm, score_v)
    pltpu.sync_copy(sidx_hbm.at[b], idx_v)
    for k in range(64):
        iv = idx_v[pl.ds(k * 16, 16)]
        row_v[pl.ds(8 + k * 16, 16)] = plsc.load_gather(score_v, [iv])

    @pl.when(s < 8)
    def _():
        _fill_pad(pad_v, row_v, rev=False)

    @pl.when(s >= 8)
    def _():
        _fill_pad(pad_v, row_v, rev=True)

    def compute_group(c0, ob):
        sj = pad_v[pl.ds(M + c0, 16)]
        aco = 5.0 - 20.0 * sj

        def sig(off):
            t = pad_v[pl.ds(off, 16)] * 20.0 + aco
            return 1.0 / (1.0 + jnp.exp(jnp.minimum(-t, 80.0)))

        def blockfn(it, run):
            i0 = it * 8
            offb = ROWS + c0 - i0
            gs = [sig(offb - u) for u in range(8)]
            a1 = gs[0] * gs[1]
            a3 = gs[2] * gs[3]
            a5 = gs[4] * gs[5]
            a7 = gs[6] * gs[7]
            b3 = a1 * a3
            p = [gs[0], a1, a1 * gs[2], b3, b3 * gs[4], b3 * a5,
                 b3 * (a5 * gs[6]), b3 * (a5 * a7)]
            for u in range(8):
                ob[i0 + u] = run * p[u]
            return run * p[7]

        v1 = 1.0 / (1.0 + jnp.exp(jnp.minimum(-aco, 80.0)))
        v2 = v1 * v1
        v4 = v2 * v2
        pw = [v1, v2, v2 * v1, v4, v4 * v1, v4 * v2, v4 * (v2 * v1), v4 * v4]

        def blockfn_c(it, run):
            i0 = it * 8
            for u in range(8):
                ob[i0 + u] = run * pw[u]
            return run * pw[7]

        nb = ROWS // 8
        ta = jnp.minimum((c0 + 22) // 8, nb)
        run = lax.fori_loop(0, ta, blockfn, jnp.ones((16,), jnp.float32))
        run = lax.fori_loop(ta, nb, blockfn_c, run)
        for i in range(8 * nb, ROWS):
            run = run * sig(ROWS + c0 - i)
            ob[i] = run

    def dma_start(ob, c0, sem):
        pltpu.async_copy(ob, out_hbm.at[dd, b, :, pl.ds(c0, 16)], sem)

    def dma_drain(ob, c0, sem):
        pltpu.make_async_copy(
            ob, out_hbm.at[dd, b, :, pl.ds(c0, 16)], sem).wait()

    def tbody(t, carry):
        c0a = (4 * t + c) * 16
        c0b = (4 * t + 2 + c) * 16

        @pl.when(t > 0)
        def _():
            dma_drain(ob0, c0a, sem0)

        compute_group(c0a, ob0)
        dma_start(ob0, c0a, sem0)

        @pl.when(t > 0)
        def _():
            dma_drain(ob1, c0b, sem1)

        compute_group(c0b, ob1)
        dma_start(ob1, c0b, sem1)
        return carry

    lax.fori_loop(0, 16, tbody, jnp.int32(0))
    dma_drain(ob0, (60 + c) * 16, sem0)
    dma_drain(ob1, (62 + c) * 16, sem1)


@functools.partial(jax.jit, static_argnames=())
def _gate_net(score, score_idx):
    mesh = plsc.VectorSubcoreMesh(core_axis_name="c", subcore_axis_name="s")
    out_ty = jax.ShapeDtypeStruct((2, B, ROWS, M), jnp.float32)
    fn = pl.kernel(
        _body,
        mesh=mesh,
        out_type=out_ty,
        scratch_types=[
            pltpu.VMEM((B * L,), jnp.float32),
            pltpu.VMEM((L,), jnp.int32),
            pltpu.VMEM((8 + L + 8,), jnp.float32),
            pltpu.VMEM((2 * L + 16,), jnp.float32),
            pltpu.VMEM((ROWS, 16), jnp.float32),
            pltpu.VMEM((ROWS, 16), jnp.float32),
            pltpu.SemaphoreType.DMA,
            pltpu.SemaphoreType.DMA,
        ],
        compiler_params=pltpu.CompilerParams(use_tc_tiling_on_sc=False,
                                             needs_layout_passes=False),
    )
    out = fn(score, score_idx)
    return out[0], out[1]


def kernel(score, rep_srcs, rep_idx, score_idx):
    del rep_srcs, rep_idx
    return _gate_net(score, score_idx.astype(jnp.int32))


# speedup vs baseline: 27.3153x; 1.2310x over previous
import functools

import jax
import jax.numpy as jnp
from jax import lax
from jax.experimental import pallas as pl
from jax.experimental.pallas import tpu as pltpu
from jax.experimental.pallas import tpu_sc as plsc

B = 8
L = 1024
M = L - 2
ROWS = M - 1


def _fill_pad(pad_v, row_v, rev):
    zz = jnp.zeros((16,), jnp.float32)
    for k in range(64):
        pad_v[pl.ds(k * 16, 16)] = zz
    for k in range(64):
        if not rev:
            v = row_v[pl.ds(9 + k * 16, 16)]
        else:
            v = lax.rev(row_v[pl.ds(1015 - k * 16, 16)], (0,))
        pad_v[pl.ds(M + k * 16, 16)] = v
    pad_v[pl.ds(M + 1022, 16)] = zz


def _body(score_hbm, sidx_hbm, fwd_hbm, bwd_hbm,
          score_v, idx_v, row_v, pad_v, ob0, ob1, sem0, sem1):
    c = lax.axis_index("c")
    s = lax.axis_index("s")
    b = jnp.where(s < 8, s, s - 8)

    pltpu.sync_copy(score_hbm, score_v)
    pltpu.sync_copy(sidx_hbm.at[b], idx_v)
    for k in range(64):
        iv = idx_v[pl.ds(k * 16, 16)]
        row_v[pl.ds(8 + k * 16, 16)] = plsc.load_gather(score_v, [iv])

    def compute_group(c0, ob):
        sj = pad_v[pl.ds(M + c0, 16)]
        aco = 5.0 - 20.0 * sj

        def sig(off):
            t = pad_v[pl.ds(off, 16)] * 20.0 + aco
            return 1.0 / (1.0 + jnp.exp(jnp.minimum(-t, 80.0)))

        def blockfn(it, run):
            i0 = it * 8
            offb = ROWS + c0 - i0
            gs = [sig(offb - u) for u in range(8)]
            a1 = gs[0] * gs[1]
            a3 = gs[2] * gs[3]
            a5 = gs[4] * gs[5]
            a7 = gs[6] * gs[7]
            b3 = a1 * a3
            p = [gs[0], a1, a1 * gs[2], b3, b3 * gs[4], b3 * a5,
                 b3 * (a5 * gs[6]), b3 * (a5 * a7)]
            for u in range(8):
                ob[i0 + u] = run * p[u]
            return run * p[7]

        v1 = 1.0 / (1.0 + jnp.exp(jnp.minimum(-aco, 80.0)))
        v2 = v1 * v1
        v4 = v2 * v2
        pw = [v1, v2, v2 * v1, v4, v4 * v1, v4 * v2, v4 * (v2 * v1), v4 * v4]

        def blockfn_c(it, run):
            i0 = it * 8
            for u in range(8):
                ob[i0 + u] = run * pw[u]
            return run * pw[7]

        nb = ROWS // 8
        ta = jnp.minimum((c0 + 22) // 8, nb)
        run = lax.fori_loop(0, ta, blockfn, jnp.ones((16,), jnp.float32))
        run = lax.fori_loop(ta, nb, blockfn_c, run)
        for i in range(8 * nb, ROWS):
            run = run * sig(ROWS + c0 - i)
            ob[i] = run

    def run_direction(out_hbm, rev):
        _fill_pad(pad_v, row_v, rev=rev)

        def dma_start(ob, c0, sem):
            pltpu.async_copy(ob, out_hbm.at[b, :, pl.ds(c0, 16)], sem)

        def dma_drain(ob, c0, sem):
            pltpu.make_async_copy(
                ob, out_hbm.at[b, :, pl.ds(c0, 16)], sem).wait()

        def tbody(t, carry):
            c0a = (4 * t + c) * 16
            c0b = (4 * t + 2 + c) * 16

            @pl.when(t > 0)
            def _():
                dma_drain(ob0, c0a, sem0)

            compute_group(c0a, ob0)
            dma_start(ob0, c0a, sem0)

            @pl.when(t > 0)
            def _():
                dma_drain(ob1, c0b, sem1)

            compute_group(c0b, ob1)
            dma_start(ob1, c0b, sem1)
            return carry

        lax.fori_loop(0, 16, tbody, jnp.int32(0))
        dma_drain(ob0, (60 + c) * 16, sem0)
        dma_drain(ob1, (62 + c) * 16, sem1)

    @pl.when(s < 8)
    def _():
        run_direction(fwd_hbm, rev=False)

    @pl.when(s >= 8)
    def _():
        run_direction(bwd_hbm, rev=True)


@functools.partial(jax.jit, static_argnames=())
def _gate_net(score, score_idx):
    mesh = plsc.VectorSubcoreMesh(core_axis_name="c", subcore_axis_name="s")
    out_ty = (jax.ShapeDtypeStruct((B, ROWS, M), jnp.float32),
              jax.ShapeDtypeStruct((B, ROWS, M), jnp.float32))
    fn = pl.kernel(
        _body,
        mesh=mesh,
        out_type=out_ty,
        scratch_types=[
            pltpu.VMEM((B * L,), jnp.float32),
            pltpu.VMEM((L,), jnp.int32),
            pltpu.VMEM((8 + L + 8,), jnp.float32),
            pltpu.VMEM((2 * L + 16,), jnp.float32),
            pltpu.VMEM((ROWS, 16), jnp.float32),
            pltpu.VMEM((ROWS, 16), jnp.float32),
            pltpu.SemaphoreType.DMA,
            pltpu.SemaphoreType.DMA,
        ],
        compiler_params=pltpu.CompilerParams(use_tc_tiling_on_sc=False,
                                             needs_layout_passes=False),
    )
    return fn(score, score_idx)


def kernel(score, rep_srcs, rep_idx, score_idx):
    del rep_srcs, rep_idx
    return _gate_net(score, score_idx.astype(jnp.int32))


# speedup vs baseline: 29.4216x; 1.0771x over previous
import functools

import jax
import jax.numpy as jnp
from jax import lax
from jax.experimental import pallas as pl
from jax.experimental.pallas import tpu as pltpu
from jax.experimental.pallas import tpu_sc as plsc

B = 8
L = 1024
M = L - 2
ROWS = M - 1


def _fill_pad(pad_v, fq_v, e_v, f_v, rev):
    one = jnp.ones((16,), jnp.float32)
    for k in range(64):
        pad_v[pl.ds(k * 16, 16)] = one
    for k in range(64):
        if not rev:
            ev = e_v[pl.ds(9 + k * 16, 16)]
            fv = f_v[pl.ds(9 + k * 16, 16)]
        else:
            ev = lax.rev(e_v[pl.ds(1015 - k * 16, 16)], (0,))
            fv = lax.rev(f_v[pl.ds(1015 - k * 16, 16)], (0,))
        pad_v[pl.ds(M + k * 16, 16)] = ev
        fq_v[pl.ds(k * 16, 16)] = fv
    pad_v[pl.ds(M + 1022, 16)] = one
    fq_v[pl.ds(M, 16)] = one


def _body(score_hbm, sidx_hbm, fwd_hbm, bwd_hbm,
          score_v, idx_v, row_v, e_v, f_v, pad_v, fq_v,
          ob0, ob1, sem0, sem1):
    c = lax.axis_index("c")
    s = lax.axis_index("s")
    b = jnp.where(s < 8, s, s - 8)

    pltpu.sync_copy(score_hbm, score_v)
    pltpu.sync_copy(sidx_hbm.at[b], idx_v)
    for k in range(64):
        iv = idx_v[pl.ds(k * 16, 16)]
        row_v[pl.ds(8 + k * 16, 16)] = plsc.load_gather(score_v, [iv])

    for k in range(64):
        rv = row_v[pl.ds(8 + k * 16, 16)]
        e_v[pl.ds(8 + k * 16, 16)] = jnp.exp(
            jnp.clip(-5.0 * rv, -85.0, 85.0))
        f_v[pl.ds(8 + k * 16, 16)] = jnp.exp(
            jnp.clip(5.0 * rv - 1.25, -85.0, 85.0))

    def compute_group(c0, ob):
        fj = fq_v[pl.ds(c0, 16)]

        def fac(off):
            g = pad_v[pl.ds(off, 16)] * fj
            g2 = g * g
            return 1.0 / (1.0 + g2 * g2)

        def blockfn(it, run):
            i0 = it * 8
            offb = ROWS + c0 - i0
            gs = [fac(offb - u) for u in range(8)]
            a1 = gs[0] * gs[1]
            a3 = gs[2] * gs[3]
            a5 = gs[4] * gs[5]
            a7 = gs[6] * gs[7]
            b3 = a1 * a3
            p = [gs[0], a1, a1 * gs[2], b3, b3 * gs[4], b3 * a5,
                 b3 * (a5 * gs[6]), b3 * (a5 * a7)]
            for u in range(8):
                ob[i0 + u] = run * p[u]
            return run * p[7]

        f2 = fj * fj
        v1 = 1.0 / (1.0 + f2 * f2)
        v2 = v1 * v1
        v4 = v2 * v2
        pw = [v1, v2, v2 * v1, v4, v4 * v1, v4 * v2, v4 * (v2 * v1), v4 * v4]

        def blockfn_c(it, run):
            i0 = it * 8
            for u in range(8):
                ob[i0 + u] = run * pw[u]
            return run * pw[7]

        nb = ROWS // 8
        ta = jnp.minimum((c0 + 22) // 8, nb)
        run = lax.fori_loop(0, ta, blockfn, jnp.ones((16,), jnp.float32))
        run = lax.fori_loop(ta, nb, blockfn_c, run)
        for i in range(8 * nb, ROWS):
            run = run * fac(ROWS + c0 - i)
            ob[i] = run

    def run_direction(out_hbm, rev):
        _fill_pad(pad_v, fq_v, e_v, f_v, rev=rev)

        def dma_start(ob, c0, sem):
            pltpu.async_copy(ob, out_hbm.at[b, :, pl.ds(c0, 16)], sem)

        def dma_drain(ob, c0, sem):
            pltpu.make_async_copy(
                ob, out_hbm.at[b, :, pl.ds(c0, 16)], sem).wait()

        def tbody(t, carry):
            c0a = (4 * t + c) * 16
            c0b = (4 * t + 2 + c) * 16

            @pl.when(t > 0)
            def _():
                dma_drain(ob0, c0a, sem0)

            compute_group(c0a, ob0)
            dma_start(ob0, c0a, sem0)

            @pl.when(t > 0)
            def _():
                dma_drain(ob1, c0b, sem1)

            compute_group(c0b, ob1)
            dma_start(ob1, c0b, sem1)
            return carry

        lax.fori_loop(0, 16, tbody, jnp.int32(0))
        dma_drain(ob0, (60 + c) * 16, sem0)
        dma_drain(ob1, (62 + c) * 16, sem1)

    @pl.when(s < 8)
    def _():
        run_direction(fwd_hbm, rev=False)

    @pl.when(s >= 8)
    def _():
        run_direction(bwd_hbm, rev=True)


@functools.partial(jax.jit, static_argnames=())
def _gate_net(score, score_idx):
    mesh = plsc.VectorSubcoreMesh(core_axis_name="c", subcore_axis_name="s")
    out_ty = (jax.ShapeDtypeStruct((B, ROWS, M), jnp.float32),
              jax.ShapeDtypeStruct((B, ROWS, M), jnp.float32))
    fn = pl.kernel(
        _body,
        mesh=mesh,
        out_type=out_ty,
        scratch_types=[
            pltpu.VMEM((B * L,), jnp.float32),
            pltpu.VMEM((L,), jnp.int32),
            pltpu.VMEM((8 + L + 8,), jnp.float32),
            pltpu.VMEM((8 + L + 8,), jnp.float32),
            pltpu.VMEM((8 + L + 8,), jnp.float32),
            pltpu.VMEM((2 * L + 16,), jnp.float32),
            pltpu.VMEM((L + 16,), jnp.float32),
            pltpu.VMEM((ROWS, 16), jnp.float32),
            pltpu.VMEM((ROWS, 16), jnp.float32),
            pltpu.SemaphoreType.DMA,
            pltpu.SemaphoreType.DMA,
        ],
        compiler_params=pltpu.CompilerParams(use_tc_tiling_on_sc=False,
                                             needs_layout_passes=False),
    )
    return fn(score, score_idx)


def kernel(score, rep_srcs, rep_idx, score_idx):
    del rep_srcs, rep_idx
    return _gate_net(score, score_idx.astype(jnp.int32))
